# baseline (speedup 1.0000x reference)
import functools

import jax
import jax.numpy as jnp
from jax import lax
from jax.experimental import pallas as pl
from jax.experimental.pallas import tpu as pltpu
from jax.experimental.pallas import tpu_sc as plsc

_D = 64
_NC = 2
_NS = 16
_CH = 4
_L = 128

_mesh = plsc.VectorSubcoreMesh(core_axis_name="c", subcore_axis_name="s", num_cores=_NC)


@functools.partial(
    pl.kernel,
    mesh=_mesh,
    compiler_params=pltpu.CompilerParams(use_tc_tiling_on_sc=False),
    out_type=jax.ShapeDtypeStruct((16, 1024, _D), jnp.float32),
    scratch_types=[
        pltpu.VMEM((_CH, _L), jnp.int32),
        pltpu.VMEM((_CH, _L, _D), jnp.float32),
    ]
    + [pltpu.SemaphoreType.DMA] * (2 * _CH),
)
def _gather_rows(idx_hbm, table_hbm, out_hbm, idx_v, rows_v, *sems):
    gsems, wsems = sems[:_CH], sems[_CH:]
    wid = lax.axis_index("s") * _NC + lax.axis_index("c")
    slab = wid // 2
    half = wid % 2
    rt = slab // 8
    ct = slab % 8
    pltpu.sync_copy(idx_hbm.at[rt, ct, pl.ds(half * _CH, _CH)], idx_v)
    gathers = [
        pltpu.async_copy(table_hbm.at[idx_v.at[j]], rows_v.at[j], gsems[j])
        for j in range(_CH)
    ]
    writes = []
    for j in range(_CH):
        gathers[j].wait()
        b = rt * 8 + half * _CH + j
        writes.append(
            pltpu.async_copy(
                rows_v.at[j], out_hbm.at[b, pl.ds(ct * _L, _L)], wsems[j]
            )
        )
    for w in writes:
        w.wait()


def _retile_body(in_ref, out_ref):
    x = in_ref[...].reshape(16, 512, 2 * _D)
    a = x[:, :, :_D].astype(jnp.bfloat16)
    b = x[:, :, _D:].astype(jnp.bfloat16)
    i_s = jax.lax.broadcasted_iota(jnp.int32, (512, 1024), 0)
    i_t = jax.lax.broadcasted_iota(jnp.int32, (512, 1024), 1)
    ea = (i_t == 2 * i_s).astype(jnp.bfloat16)
    eb = (i_t == 2 * i_s + 1).astype(jnp.bfloat16)

    def dg(m, e):
        return jax.lax.dot_general(
            m,
            e,
            (((1,), (0,)), ((), ())),
            precision=jax.lax.Precision.DEFAULT,
            preferred_element_type=jnp.float32,
        )

    out_ref[...] = dg(a, ea) + dg(b, eb)


_retile = pl.pallas_call(
    _retile_body,
    out_shape=jax.ShapeDtypeStruct((16, _D, 1024), jnp.float32),
)


def kernel(embed_id, weight):
    idx4 = jnp.transpose(
        embed_id.astype(jnp.int32).reshape(2, 8, 8, 128), (0, 2, 1, 3)
    )
    out_sc = _gather_rows(idx4, weight)
    out_bdt = _retile(out_sc.reshape(-1))
    return jnp.transpose(out_bdt, (0, 2, 1))


# speedup vs baseline: 1.0115x; 1.0115x over previous
import functools

import jax
import jax.numpy as jnp
from jax import lax
from jax.experimental import pallas as pl
from jax.experimental.pallas import tpu as pltpu
from jax.experimental.pallas import tpu_sc as plsc

_D = 64
_NC = 2
_NS = 16
_CH = 4
_L = 128

_mesh = plsc.VectorSubcoreMesh(core_axis_name="c", subcore_axis_name="s", num_cores=_NC)


@functools.partial(
    pl.kernel,
    mesh=_mesh,
    compiler_params=pltpu.CompilerParams(use_tc_tiling_on_sc=False),
    out_type=jax.ShapeDtypeStruct((16, 1024, _D), jnp.float32),
    scratch_types=[
        pltpu.VMEM((_CH, _L), jnp.int32),
        pltpu.VMEM((_CH, _L, _D), jnp.float32),
    ]
    + [pltpu.SemaphoreType.DMA] * (2 * _CH),
)
def _gather_rows(idx_hbm, table_hbm, out_hbm, idx_v, rows_v, *sems):
    gsems, wsems = sems[:_CH], sems[_CH:]
    wid = lax.axis_index("s") * _NC + lax.axis_index("c")
    slab = wid // 2
    half = wid % 2
    rt = slab // 8
    ct = slab % 8
    pltpu.sync_copy(idx_hbm.at[rt, ct, pl.ds(half * _CH, _CH)], idx_v)
    gathers = [
        pltpu.async_copy(table_hbm.at[idx_v.at[j]], rows_v.at[j], gsems[j])
        for j in range(_CH)
    ]
    writes = []
    for j in range(_CH):
        gathers[j].wait()
        b = rt * 8 + half * _CH + j
        writes.append(
            pltpu.async_copy(
                rows_v.at[j], out_hbm.at[b, pl.ds(ct * _L, _L)], wsems[j]
            )
        )
    for w in writes:
        w.wait()


def _retile_body(in_ref, out_ref):
    x = in_ref[...].reshape(4, 512, 2 * _D)
    a = x[:, :, :_D]
    b = x[:, :, _D:]
    i_s = jax.lax.broadcasted_iota(jnp.int32, (512, 1024), 0)
    i_t = jax.lax.broadcasted_iota(jnp.int32, (512, 1024), 1)
    ea = (i_t == 2 * i_s).astype(jnp.float32)
    eb = (i_t == 2 * i_s + 1).astype(jnp.float32)

    def dg(m, e):
        return jax.lax.dot_general(
            m,
            e,
            (((1,), (0,)), ((), ())),
            precision=jax.lax.Precision.DEFAULT,
            preferred_element_type=jnp.float32,
        )

    out_ref[...] = dg(a, ea) + dg(b, eb)


_retile = pl.pallas_call(
    _retile_body,
    grid=(4,),
    in_specs=[pl.BlockSpec((4 * 512 * 2 * _D,), lambda g: (g,))],
    out_specs=pl.BlockSpec((4, _D, 1024), lambda g: (g, 0, 0)),
    out_shape=jax.ShapeDtypeStruct((16, _D, 1024), jnp.float32),
)


def kernel(embed_id, weight):
    idx4 = jnp.transpose(
        embed_id.astype(jnp.int32).reshape(2, 8, 8, 128), (0, 2, 1, 3)
    )
    out_sc = _gather_rows(idx4, weight)
    out_bdt = _retile(out_sc.reshape(-1))
    return jnp.transpose(out_bdt, (0, 2, 1))
